# Initial kernel scaffold; baseline (speedup 1.0000x reference)
import jax
import jax.numpy as jnp
from jax.experimental import pallas as pl


def kernel(x, table):
    raise NotImplementedError("write your pallas kernel here")


# speedup vs baseline: 3.7589x; 3.7589x over previous
import functools

import jax
import jax.numpy as jnp
from jax import lax
from jax.experimental import pallas as pl
from jax.experimental.pallas import tpu as pltpu
from jax.experimental.pallas import tpu_sc as plsc

_VOCAB = 100000
_D = 64
_B = 4096 * 200

_NC = 2
_NS = 16
_NW = _NC * _NS
_PER_W = _B // _NW

_IDXW = 128
_CHUNK = 1024
_STREAMS = _CHUNK // _IDXW
_NCHUNK = _PER_W // _CHUNK


def _scale_body(t_ref, o_ref):
    o_ref[...] = t_ref[...] * 8.0


@jax.jit
def _scale_table(table):
    rows = 2000
    return pl.pallas_call(
        _scale_body,
        grid=(_VOCAB // rows,),
        in_specs=[pl.BlockSpec((rows, _D), lambda i: (i, 0))],
        out_specs=pl.BlockSpec((rows, _D), lambda i: (i, 0)),
        out_shape=jax.ShapeDtypeStruct((_VOCAB, _D), jnp.float32),
    )(table)


def _gather_body(table_hbm, idx_hbm, out_hbm, idx_v, rows_v, sem):
    wid = lax.axis_index("s") * _NC + lax.axis_index("c")
    idx_row_base = wid * (_PER_W // _IDXW)
    out_base = wid * _PER_W

    for g in range(_NCHUNK):
        pltpu.sync_copy(idx_hbm.at[pl.ds(idx_row_base + g * _STREAMS, _STREAMS)],
                        idx_v)
        copies = []
        for j in range(_STREAMS):
            copies.append(
                pltpu.async_copy(table_hbm.at[idx_v.at[j]],
                                 rows_v.at[pl.ds(j * _IDXW, _IDXW)],
                                 sem))
        for c in copies:
            c.wait()
        pltpu.sync_copy(rows_v,
                        out_hbm.at[pl.ds(out_base + g * _CHUNK, _CHUNK)])


@jax.jit
def _sc_gather(table, idx2d):
    mesh = plsc.VectorSubcoreMesh(core_axis_name="c", subcore_axis_name="s")
    return pl.kernel(
        _gather_body,
        out_type=jax.ShapeDtypeStruct((_B, _D), jnp.float32),
        mesh=mesh,
        scratch_types=[
            pltpu.VMEM((_STREAMS, _IDXW), jnp.int32),
            pltpu.VMEM((_CHUNK, _D), jnp.float32),
            pltpu.SemaphoreType.DMA,
        ],
        compiler_params=pltpu.CompilerParams(use_tc_tiling_on_sc=False),
    )(table, idx2d)


def kernel(x, table):
    idx2d = x.reshape(_B // _IDXW, _IDXW)
    scaled = _scale_table(table)
    out = _sc_gather(scaled, idx2d)
    return out.reshape(x.shape[0], x.shape[1], _D)


# speedup vs baseline: 3.8653x; 1.0283x over previous
import functools

import jax
import jax.numpy as jnp
from jax import lax
from jax.experimental import pallas as pl
from jax.experimental.pallas import tpu as pltpu
from jax.experimental.pallas import tpu_sc as plsc

_VOCAB = 100000
_D = 64
_B = 4096 * 200

_NC = 2
_NS = 16
_NW = _NC * _NS
_PER_W = _B // _NW

_IDXW = 128
_CHUNK = 512
_STREAMS = _CHUNK // _IDXW
_NCHUNK = _PER_W // _CHUNK
_NBUF = 2


def _scale_body(t_ref, o_ref):
    o_ref[...] = t_ref[...] * 8.0


@jax.jit
def _scale_table(table):
    rows = 2000
    return pl.pallas_call(
        _scale_body,
        grid=(_VOCAB // rows,),
        in_specs=[pl.BlockSpec((rows, _D), lambda i: (i, 0))],
        out_specs=pl.BlockSpec((rows, _D), lambda i: (i, 0)),
        out_shape=jax.ShapeDtypeStruct((_VOCAB, _D), jnp.float32),
    )(table)


def _gather_body(table_hbm, idx_hbm, out_hbm, idx_v, rows_v, gsems, osems):
    wid = lax.axis_index("s") * _NC + lax.axis_index("c")
    idx_row_base = wid * (_PER_W // _IDXW)
    out_base = wid * _PER_W

    pltpu.sync_copy(idx_hbm.at[pl.ds(idx_row_base, _PER_W // _IDXW)], idx_v)

    def fire_gathers(g, b):
        copies = []
        for j in range(_STREAMS):
            copies.append(
                pltpu.async_copy(table_hbm.at[idx_v.at[g * _STREAMS + j]],
                                 rows_v.at[b].at[pl.ds(j * _IDXW, _IDXW)],
                                 gsems.at[b]))
        return copies

    out_copies = [None] * _NBUF
    gathers = fire_gathers(0, 0)
    for g in range(_NCHUNK):
        b = g % _NBUF
        nb = (g + 1) % _NBUF
        if g + 1 < _NCHUNK:
            if out_copies[nb] is not None:
                out_copies[nb].wait()
            next_gathers = fire_gathers(g + 1, nb)
        for c in gathers:
            c.wait()
        out_copies[b] = pltpu.async_copy(
            rows_v.at[b],
            out_hbm.at[pl.ds(out_base + g * _CHUNK, _CHUNK)],
            osems.at[b])
        if g + 1 < _NCHUNK:
            gathers = next_gathers
    for c in out_copies:
        if c is not None:
            c.wait()


@jax.jit
def _sc_gather(table, idx2d):
    mesh = plsc.VectorSubcoreMesh(core_axis_name="c", subcore_axis_name="s")
    return pl.kernel(
        _gather_body,
        out_type=jax.ShapeDtypeStruct((_B, _D), jnp.float32),
        mesh=mesh,
        scratch_types=[
            pltpu.VMEM((_PER_W // _IDXW, _IDXW), jnp.int32),
            pltpu.VMEM((_NBUF, _CHUNK, _D), jnp.float32),
            pltpu.SemaphoreType.DMA((_NBUF,)),
            pltpu.SemaphoreType.DMA((_NBUF,)),
        ],
        compiler_params=pltpu.CompilerParams(use_tc_tiling_on_sc=False),
    )(table, idx2d)


def kernel(x, table):
    idx2d = x.reshape(_B // _IDXW, _IDXW)
    scaled = _scale_table(table)
    out = _sc_gather(scaled, idx2d)
    return out.reshape(x.shape[0], x.shape[1], _D)


# speedup vs baseline: 4.2157x; 1.0907x over previous
import functools

import jax
import jax.numpy as jnp
from jax import lax
from jax.experimental import pallas as pl
from jax.experimental.pallas import tpu as pltpu
from jax.experimental.pallas import tpu_sc as plsc

_VOCAB = 100000
_D = 64
_B = 4096 * 200

_NC = 2
_NS = 16
_NW = _NC * _NS
_PER_W = _B // _NW

_IDXW = 128
_CHUNK = 512
_STREAMS = _CHUNK // _IDXW
_NCHUNK = _PER_W // _CHUNK
_NBUF = 2
_SCALE = 8.0


def _gather_body(table_hbm, idx_hbm, out_hbm, idx_v, rows_v, gsems, osems):
    wid = lax.axis_index("s") * _NC + lax.axis_index("c")
    idx_row_base = wid * (_PER_W // _IDXW)
    out_base = wid * _PER_W

    pltpu.sync_copy(idx_hbm.at[pl.ds(idx_row_base, _PER_W // _IDXW)], idx_v)

    def fire_gathers(g, b):
        copies = []
        for j in range(_STREAMS):
            copies.append(
                pltpu.async_copy(table_hbm.at[idx_v.at[g * _STREAMS + j]],
                                 rows_v.at[b].at[pl.ds(j * _IDXW, _IDXW)],
                                 gsems.at[b]))
        return copies

    def scale_chunk(b):
        rv = rows_v.at[b]

        @plsc.parallel_loop(0, _CHUNK, unroll=8)
        def _(i):
            for j in range(_D // 16):
                s = pl.ds(j * 16, 16)
                rv[i, s] = rv[i, s] * _SCALE

    out_copies = [None] * _NBUF
    gathers = fire_gathers(0, 0)
    for g in range(_NCHUNK):
        b = g % _NBUF
        nb = (g + 1) % _NBUF
        if g + 1 < _NCHUNK:
            if out_copies[nb] is not None:
                out_copies[nb].wait()
            next_gathers = fire_gathers(g + 1, nb)
        for c in gathers:
            c.wait()
        scale_chunk(b)
        out_copies[b] = pltpu.async_copy(
            rows_v.at[b],
            out_hbm.at[pl.ds(out_base + g * _CHUNK, _CHUNK)],
            osems.at[b])
        if g + 1 < _NCHUNK:
            gathers = next_gathers
    for c in out_copies:
        if c is not None:
            c.wait()


@jax.jit
def _sc_gather(table, idx2d):
    mesh = plsc.VectorSubcoreMesh(core_axis_name="c", subcore_axis_name="s")
    return pl.kernel(
        _gather_body,
        out_type=jax.ShapeDtypeStruct((_B, _D), jnp.float32),
        mesh=mesh,
        scratch_types=[
            pltpu.VMEM((_PER_W // _IDXW, _IDXW), jnp.int32),
            pltpu.VMEM((_NBUF, _CHUNK, _D), jnp.float32),
            pltpu.SemaphoreType.DMA((_NBUF,)),
            pltpu.SemaphoreType.DMA((_NBUF,)),
        ],
        compiler_params=pltpu.CompilerParams(use_tc_tiling_on_sc=False),
    )(table, idx2d)


def kernel(x, table):
    idx2d = x.reshape(_B // _IDXW, _IDXW)
    out = _sc_gather(table, idx2d)
    return out.reshape(x.shape[0], x.shape[1], _D)
